# baseline (speedup 1.0000x reference)
import functools

import jax
import jax.numpy as jnp
from jax import lax
from jax.experimental import pallas as pl
from jax.experimental.pallas import tpu as pltpu
from jax.experimental.pallas import tpu_sc as plsc

N_USERS = 1000000
N_ITEMS = 1000000
EMBED_DIM = 64
BATCH = 16384

_info = plsc.get_sparse_core_info()
NC, NS, L = _info.num_cores, _info.num_subcores, _info.num_lanes
NW = NC * NS
B_PER_W = BATCH // NW
IDX_CHUNK = 128
N_CHUNKS = B_PER_W // IDX_CHUNK
GROUPS = B_PER_W // L


def _mf_kernel(user_hbm, item_hbm, uemb_hbm, iemb_hbm, out_hbm,
               idx_u, idx_i, rows_u, rows_i, out_v, sem):
    wid = lax.axis_index("s") * NC + lax.axis_index("c")
    base = wid * B_PER_W

    for j in range(N_CHUNKS):
        pltpu.sync_copy(user_hbm.at[pl.ds(base + j * IDX_CHUNK, IDX_CHUNK)],
                        idx_u.at[j])
        pltpu.sync_copy(item_hbm.at[pl.ds(base + j * IDX_CHUNK, IDX_CHUNK)],
                        idx_i.at[j])

    copies = []
    for j in range(N_CHUNKS):
        copies.append(pltpu.async_copy(
            uemb_hbm.at[idx_u.at[j]],
            rows_u.at[pl.ds(j * IDX_CHUNK, IDX_CHUNK)], sem))
        copies.append(pltpu.async_copy(
            iemb_hbm.at[idx_i.at[j]],
            rows_i.at[pl.ds(j * IDX_CHUNK, IDX_CHUNK)], sem))
    for c in copies:
        c.wait()

    lane = lax.iota(jnp.int32, L)

    def group_body(g, carry):
        base_b = g * L
        res = jnp.zeros((L,), jnp.float32)
        for e in range(L):
            b = base_b + e
            acc = rows_u[b, pl.ds(0, L)] * rows_i[b, pl.ds(0, L)]
            for j in range(1, EMBED_DIM // L):
                acc = acc + rows_u[b, pl.ds(j * L, L)] * rows_i[b, pl.ds(j * L, L)]
            res = jnp.where(lane == e, jnp.sum(acc), res)
        out_v[pl.ds(base_b, L)] = 1.0 / (1.0 + jnp.exp(-res))
        return carry

    lax.fori_loop(0, GROUPS, group_body, 0)

    pltpu.sync_copy(out_v, out_hbm.at[pl.ds(base, B_PER_W)])


@functools.partial(jax.jit, static_argnums=())
def kernel(user, item, user_embed, item_embed):
    user = user.astype(jnp.int32)
    item = item.astype(jnp.int32)
    mesh = plsc.VectorSubcoreMesh(core_axis_name="c", subcore_axis_name="s")
    f = pl.kernel(
        _mf_kernel,
        mesh=mesh,
        compiler_params=pltpu.CompilerParams(
            needs_layout_passes=False, use_tc_tiling_on_sc=False),
        out_type=jax.ShapeDtypeStruct((BATCH,), jnp.float32),
        scratch_types=[
            pltpu.VMEM((N_CHUNKS, IDX_CHUNK), jnp.int32),
            pltpu.VMEM((N_CHUNKS, IDX_CHUNK), jnp.int32),
            pltpu.VMEM((B_PER_W, EMBED_DIM), jnp.float32),
            pltpu.VMEM((B_PER_W, EMBED_DIM), jnp.float32),
            pltpu.VMEM((B_PER_W,), jnp.float32),
            pltpu.SemaphoreType.DMA,
        ],
    )
    return f(user, item, user_embed, item_embed)


# speedup vs baseline: 2.3611x; 2.3611x over previous
import jax
import jax.numpy as jnp
from jax import lax
from jax.experimental import pallas as pl
from jax.experimental.pallas import tpu as pltpu
from jax.experimental.pallas import tpu_sc as plsc

EMBED_DIM = 64
BATCH = 16384
N_ROWS = 1000001
LANES = 128

_info = plsc.get_sparse_core_info()
NC, NS, L = _info.num_cores, _info.num_subcores, _info.num_lanes
NW = NC * NS
B_PER_W = BATCH // NW
EPB = 4
NBATCH = B_PER_W // EPB


def _mf_kernel(user_hbm, item_hbm, uemb_hbm, iemb_hbm, out_hbm,
               idx_u, idx_i, buf_u, buf_i, out_v, sem):
    wid = lax.axis_index("s") * NC + lax.axis_index("c")
    base = wid * B_PER_W

    pltpu.sync_copy(user_hbm.at[pl.ds(base, B_PER_W)], idx_u)
    pltpu.sync_copy(item_hbm.at[pl.ds(base, B_PER_W)], idx_i)

    lane = lax.iota(jnp.int32, L)
    dvs = [j * L + lane for j in range(EMBED_DIM // L)]

    def group_body(g, carry):
        base_g = g * L
        uvec = idx_u[pl.ds(base_g, L)]
        ivec = idx_i[pl.ds(base_g, L)]
        res = jnp.zeros((L,), jnp.float32)
        for sb in range(L // EPB):
            copies = []
            lanes_u = []
            lanes_i = []
            for e in range(EPB):
                ru = uvec[sb * EPB + e]
                ri = ivec[sb * EPB + e]
                copies.append(pltpu.async_copy(
                    uemb_hbm.at[:, pl.ds((ru // LANES) * LANES, LANES)],
                    buf_u.at[e], sem))
                copies.append(pltpu.async_copy(
                    iemb_hbm.at[:, pl.ds((ri // LANES) * LANES, LANES)],
                    buf_i.at[e], sem))
                lanes_u.append(jnp.full((L,), ru % LANES, jnp.int32))
                lanes_i.append(jnp.full((L,), ri % LANES, jnp.int32))
            for c in copies:
                c.wait()
            for e in range(EPB):
                acc = jnp.zeros((L,), jnp.float32)
                for j in range(EMBED_DIM // L):
                    gu = plsc.load_gather(buf_u.at[e], [dvs[j], lanes_u[e]])
                    gi = plsc.load_gather(buf_i.at[e], [dvs[j], lanes_i[e]])
                    acc = acc + gu * gi
                res = jnp.where(lane == sb * EPB + e, jnp.sum(acc), res)
        out_v[pl.ds(base_g, L)] = 1.0 / (1.0 + jnp.exp(-res))
        return carry

    lax.fori_loop(0, B_PER_W // L, group_body, 0)

    pltpu.sync_copy(out_v, out_hbm.at[pl.ds(base, B_PER_W)])


@jax.jit
def kernel(user, item, user_embed, item_embed):
    user = user.astype(jnp.int32)
    item = item.astype(jnp.int32)
    uemb_t = user_embed.T
    iemb_t = item_embed.T
    mesh = plsc.VectorSubcoreMesh(core_axis_name="c", subcore_axis_name="s")
    f = pl.kernel(
        _mf_kernel,
        mesh=mesh,
        compiler_params=pltpu.CompilerParams(needs_layout_passes=False),
        out_type=jax.ShapeDtypeStruct((BATCH,), jnp.float32),
        scratch_types=[
            pltpu.VMEM((B_PER_W,), jnp.int32),
            pltpu.VMEM((B_PER_W,), jnp.int32),
            pltpu.VMEM((EPB, EMBED_DIM, LANES), jnp.float32),
            pltpu.VMEM((EPB, EMBED_DIM, LANES), jnp.float32),
            pltpu.VMEM((B_PER_W,), jnp.float32),
            pltpu.SemaphoreType.DMA,
        ],
    )
    return f(user, item, uemb_t, iemb_t)


# speedup vs baseline: 2.8805x; 1.2200x over previous
import jax
import jax.numpy as jnp
from jax import lax
from jax.experimental import pallas as pl
from jax.experimental.pallas import tpu as pltpu
from jax.experimental.pallas import tpu_sc as plsc

EMBED_DIM = 64
BATCH = 16384
N_ROWS = 1000001
LANES = 128

_info = plsc.get_sparse_core_info()
NC, NS, L = _info.num_cores, _info.num_subcores, _info.num_lanes
NW = NC * NS
B_PER_W = BATCH // NW
EPB = 2
SLOTS = 2


def _mf_kernel(user_hbm, item_hbm, uemb_hbm, iemb_hbm, out_hbm,
               idx_u, idx_i, buf_u, buf_i, out_v, sem):
    wid = lax.axis_index("s") * NC + lax.axis_index("c")
    base = wid * B_PER_W

    pltpu.sync_copy(user_hbm.at[pl.ds(base, B_PER_W)], idx_u)
    pltpu.sync_copy(item_hbm.at[pl.ds(base, B_PER_W)], idx_i)

    lane = lax.iota(jnp.int32, L)
    dvs = [j * L + lane for j in range(EMBED_DIM // L)]

    NSB = L // EPB

    def group_body(g, carry):
        base_g = g * L
        uvec = idx_u[pl.ds(base_g, L)]
        ivec = idx_i[pl.ds(base_g, L)]

        def issue(sb, slot):
            copies = []
            lanes_u = []
            lanes_i = []
            for e in range(EPB):
                ru = uvec[sb * EPB + e]
                ri = ivec[sb * EPB + e]
                copies.append(pltpu.async_copy(
                    uemb_hbm.at[:, pl.ds((ru // LANES) * LANES, LANES)],
                    buf_u.at[slot * EPB + e], sem))
                copies.append(pltpu.async_copy(
                    iemb_hbm.at[:, pl.ds((ri // LANES) * LANES, LANES)],
                    buf_i.at[slot * EPB + e], sem))
                lanes_u.append(jnp.full((L,), ru % LANES, jnp.int32))
                lanes_i.append(jnp.full((L,), ri % LANES, jnp.int32))
            return copies, lanes_u, lanes_i

        res = jnp.zeros((L,), jnp.float32)
        pending = issue(0, 0)
        for sb in range(NSB):
            slot = sb % SLOTS
            copies, lanes_u, lanes_i = pending
            if sb + 1 < NSB:
                pending = issue(sb + 1, (sb + 1) % SLOTS)
            for c in copies:
                c.wait()
            for e in range(EPB):
                acc = jnp.zeros((L,), jnp.float32)
                for j in range(EMBED_DIM // L):
                    gu = plsc.load_gather(
                        buf_u.at[slot * EPB + e], [dvs[j], lanes_u[e]])
                    gi = plsc.load_gather(
                        buf_i.at[slot * EPB + e], [dvs[j], lanes_i[e]])
                    acc = acc + gu * gi
                res = jnp.where(lane == sb * EPB + e, jnp.sum(acc), res)
        out_v[pl.ds(base_g, L)] = 1.0 / (1.0 + jnp.exp(-res))
        return carry

    lax.fori_loop(0, B_PER_W // L, group_body, 0)

    pltpu.sync_copy(out_v, out_hbm.at[pl.ds(base, B_PER_W)])


@jax.jit
def kernel(user, item, user_embed, item_embed):
    user = user.astype(jnp.int32)
    item = item.astype(jnp.int32)
    uemb_t = user_embed.T
    iemb_t = item_embed.T
    mesh = plsc.VectorSubcoreMesh(core_axis_name="c", subcore_axis_name="s")
    f = pl.kernel(
        _mf_kernel,
        mesh=mesh,
        compiler_params=pltpu.CompilerParams(needs_layout_passes=False),
        out_type=jax.ShapeDtypeStruct((BATCH,), jnp.float32),
        scratch_types=[
            pltpu.VMEM((B_PER_W,), jnp.int32),
            pltpu.VMEM((B_PER_W,), jnp.int32),
            pltpu.VMEM((SLOTS * EPB, EMBED_DIM, LANES), jnp.float32),
            pltpu.VMEM((SLOTS * EPB, EMBED_DIM, LANES), jnp.float32),
            pltpu.VMEM((B_PER_W,), jnp.float32),
            pltpu.SemaphoreType.DMA,
        ],
    )
    return f(user, item, uemb_t, iemb_t)
